# baseline (speedup 1.0000x reference)
import functools

import jax
import jax.numpy as jnp
import numpy as np
from jax import lax
from jax.experimental import pallas as pl
from jax.experimental.pallas import tpu as pltpu
from jax.experimental.pallas import tpu_sc as plsc

VOCAB = 100000
D = 128
BATCH = 4096
HIST = 50

_info = plsc.get_sparse_core_info()
NC = _info.num_cores
NS = _info.num_subcores
NW = NC * NS
LANES = 16

BPW = BATCH // NW
NBUF = 6

SCALE = float(np.sqrt(np.float32(D)))


def _gather_body(idx_hbm, table_hbm, out_hbm,
                 ix, rn0, rn1, rn2, rn3, rn4, rn5,
                 s0, s1, s2, s3, s4, s5):
    wid = lax.axis_index("s") * NC + lax.axis_index("c")
    base_b = wid * BPW

    bufs = ((rn0, s0), (rn1, s1), (rn2, s2), (rn3, s3), (rn4, s4), (rn5, s5))

    pltpu.sync_copy(idx_hbm.at[wid], ix)

    def start_gather(c, par):
        rn, sem = bufs[par]
        pltpu.async_copy(table_hbm.at[ix.at[c]], rn, sem)

    def wait_gather(c, par):
        rn, sem = bufs[par]
        pltpu.make_async_copy(table_hbm.at[ix.at[c]], rn, sem).wait()

    def out_slice(c):
        return out_hbm.at[c, pl.ds(base_b, BPW), :]

    def start_out(c, par):
        rn, sem = bufs[par]
        pltpu.async_copy(rn, out_slice(c), sem)

    def wait_out(c, par):
        rn, sem = bufs[par]
        pltpu.make_async_copy(rn, out_slice(c), sem).wait()

    def scale_chunk(par):
        rn, _ = bufs[par]

        def row_body(r, _):
            for q in range(D // LANES):
                sl = pl.ds(q * LANES, LANES)
                rn[r, sl] = rn[r, sl] * SCALE
            return 0

        lax.fori_loop(0, BPW, row_body, 0)

    def process(c, par, *, drain, issue):
        wait_gather(c, par)
        scale_chunk(par)
        start_out(c, par)
        if drain:
            wait_out(c - 1, (par + NBUF - 1) % NBUF)
        if issue:
            start_gather(c + NBUF - 1, (par + NBUF - 1) % NBUF)

    for c in range(NBUF - 1):
        start_gather(c, c)

    process(0, 0, drain=False, issue=True)
    for c in range(1, NBUF):
        process(c, c % NBUF, drain=True, issue=True)

    STEADY_ITERS = (HIST - 2 * (NBUF - 1)) // NBUF

    def steady(g, _):
        for p in range(NBUF):
            c = NBUF * g + p
            process(c, p, drain=True, issue=True)
        return 0

    lax.fori_loop(1, STEADY_ITERS, steady, 0)

    for c in range(NBUF * STEADY_ITERS, HIST):
        process(c, c % NBUF, drain=True, issue=(c + NBUF - 1 < HIST))

    wait_out(HIST - 1, (HIST - 1) % NBUF)


@jax.jit
def _embed(x, table):
    idx = x.reshape(NW, BPW, HIST).transpose(0, 2, 1)
    call = functools.partial(
        pl.kernel,
        mesh=plsc.VectorSubcoreMesh(core_axis_name="c", subcore_axis_name="s"),
        out_type=jax.ShapeDtypeStruct((HIST, BATCH, D), jnp.float32),
        scratch_types=[
            pltpu.VMEM((HIST, BPW), jnp.int32),
            pltpu.VMEM((BPW, D), jnp.float32),
            pltpu.VMEM((BPW, D), jnp.float32),
            pltpu.VMEM((BPW, D), jnp.float32),
            pltpu.VMEM((BPW, D), jnp.float32),
            pltpu.VMEM((BPW, D), jnp.float32),
            pltpu.VMEM((BPW, D), jnp.float32),
            pltpu.SemaphoreType.DMA,
            pltpu.SemaphoreType.DMA,
            pltpu.SemaphoreType.DMA,
            pltpu.SemaphoreType.DMA,
            pltpu.SemaphoreType.DMA,
            pltpu.SemaphoreType.DMA,
        ],
    )(_gather_body)
    hbd = call(idx, table)
    return jnp.transpose(hbd, (1, 0, 2))


def kernel(x, input_embedding_table):
    return _embed(x.astype(jnp.int32), input_embedding_table)
